# baseline (speedup 1.0000x reference)
import functools

import jax
import jax.numpy as jnp
from jax import lax
from jax.experimental import pallas as pl
from jax.experimental.pallas import tpu as pltpu
from jax.experimental.pallas import tpu_sc as plsc

NUM_USERS = 100000
NUM_MOVIES = 100000
D = 128
B = 16384

NC = 2
NS = 16
L = 16
NW = NC * NS
ROWS_PER_W = B // NW
CHUNK = 64
NCHUNK = ROWS_PER_W // CHUNK
NG = CHUNK // L
NBUF = 4


def _rsqrt(x):
    i = lax.bitcast_convert_type(x, jnp.int32)
    y = lax.bitcast_convert_type(
        jnp.int32(0x5F3759DF) - lax.shift_right_arithmetic(i, jnp.int32(1)),
        jnp.float32)
    for _ in range(3):
        y = y * (jnp.float32(1.5) - jnp.float32(0.5) * x * y * y)
    return y


def _bias_renorm(b):
    ab = jnp.abs(b)
    return b * jnp.where(ab > 2.0, 2.0 / (ab + 1e-7), 1.0)


_mesh = plsc.VectorSubcoreMesh(core_axis_name="c", subcore_axis_name="s")


@functools.partial(
    pl.kernel,
    mesh=_mesh,
    compiler_params=pltpu.CompilerParams(needs_layout_passes=False),
    out_type=jax.ShapeDtypeStruct((B,), jnp.float32),
    scratch_types=[
        pltpu.VMEM((ROWS_PER_W,), jnp.int32),
        pltpu.VMEM((ROWS_PER_W,), jnp.int32),
        pltpu.VMEM((NBUF, CHUNK, D), jnp.float32),
        pltpu.VMEM((NBUF, CHUNK, D), jnp.float32),
        pltpu.VMEM((ROWS_PER_W,), jnp.float32),
        pltpu.VMEM((ROWS_PER_W,), jnp.float32),
        pltpu.VMEM((2, CHUNK), jnp.float32),
        pltpu.SemaphoreType.DMA,
        pltpu.SemaphoreType.DMA,
        pltpu.SemaphoreType.DMA,
        pltpu.SemaphoreType.DMA,
        pltpu.SemaphoreType.DMA,
    ],
)
def _sc_predict(uf_hbm, mf_hbm, ub_hbm, mb_hbm, users_hbm, movies_hbm,
                out_hbm, uidx, midx, ubuf, mbuf, ubv, mbv, obuf,
                sem0, sem1, sem2, sem3, osem):
    wid = lax.axis_index("s") * NC + lax.axis_index("c")
    base = wid * ROWS_PER_W
    sems = (sem0, sem1, sem2, sem3)

    icp1 = pltpu.async_copy(users_hbm.at[pl.ds(base, ROWS_PER_W)], uidx, osem)
    icp2 = pltpu.async_copy(movies_hbm.at[pl.ds(base, ROWS_PER_W)], midx, osem)
    icp1.wait()
    icp2.wait()

    def fire(c):
        s = c % NBUF
        ui = uidx.at[pl.ds(c * CHUNK, CHUNK)]
        mi = midx.at[pl.ds(c * CHUNK, CHUNK)]
        return [
            pltpu.async_copy(uf_hbm.at[ui], ubuf.at[s], sems[s]),
            pltpu.async_copy(mf_hbm.at[mi], mbuf.at[s], sems[s]),
        ]

    bias_cps = []
    for t in range(ROWS_PER_W // 128):
        sl = pl.ds(t * 128, 128)
        bias_cps.append(pltpu.async_copy(
            ub_hbm.at[uidx.at[sl]], ubv.at[sl], sems[t % NBUF]))
        bias_cps.append(pltpu.async_copy(
            mb_hbm.at[midx.at[sl]], mbv.at[sl], sems[t % NBUF]))

    inflight = [fire(0), fire(1), fire(2)]
    out_cps = []
    for c in range(NCHUNK):
        s = c % NBUF
        if c == 0:
            for cp in bias_cps:
                cp.wait()
        for cp in inflight.pop(0):
            cp.wait()
        if c + 3 < NCHUNK:
            inflight.append(fire(c + 3))
        if c >= 2:
            out_cps[c - 2].wait()

        ub2 = ubuf.at[s]
        mb2 = mbuf.at[s]
        lanei = lax.broadcasted_iota(jnp.int32, (L,), 0)
        rowss = [jnp.int32(g * L) + lanei for g in range(NG)]
        zero = jnp.zeros((L,), jnp.float32)

        def col_body(j, carry):
            cols = jnp.bitwise_and(lanei + j, jnp.int32(D - 1))
            out = []
            for g in range(NG):
                ad, au, am = carry[g]
                u = plsc.load_gather(ub2, [rowss[g], cols])
                m = plsc.load_gather(mb2, [rowss[g], cols])
                out.append((ad + u * m, au + u * u, am + m * m))
            return tuple(out)

        accs = lax.fori_loop(0, D, col_body,
                             tuple((zero, zero, zero) for _ in range(NG)))

        for g in range(NG):
            ad, au, am = accs[g]
            q = jnp.maximum(au * am, jnp.float32(1e-30))
            inv = _rsqrt(q)
            p = q * inv
            cos = jnp.where(p >= 1e-8, ad * inv, ad * jnp.float32(1e8))
            bu = _bias_renorm(ubv[pl.ds(c * CHUNK + g * L, L)])
            bm = _bias_renorm(mbv[pl.ds(c * CHUNK + g * L, L)])
            pred = (cos + bu + bm) * jnp.float32(2.25) + jnp.float32(2.75)
            obuf[c % 2, pl.ds(g * L, L)] = jnp.clip(pred, 0.5, 5.0)

        out_cps.append(pltpu.async_copy(
            obuf.at[c % 2], out_hbm.at[pl.ds(base + c * CHUNK, CHUNK)], osem))
    for cp in out_cps[-2:]:
        cp.wait()


def kernel(user_factors, movie_factors, user_biases, movie_biases, users, movies):
    return _sc_predict(user_factors, movie_factors,
                       user_biases[:, 0], movie_biases[:, 0],
                       users, movies)


# speedup vs baseline: 1.0277x; 1.0277x over previous
import functools

import jax
import jax.numpy as jnp
from jax import lax
from jax.experimental import pallas as pl
from jax.experimental.pallas import tpu as pltpu
from jax.experimental.pallas import tpu_sc as plsc

NUM_USERS = 100000
NUM_MOVIES = 100000
D = 128
B = 16384

NC = 2
NS = 16
L = 16
NW = NC * NS
ROWS_PER_W = B // NW
CHUNK = 64
NCHUNK = ROWS_PER_W // CHUNK
NG = CHUNK // L
NBUF = 4


def _rsqrt(x):
    i = lax.bitcast_convert_type(x, jnp.int32)
    y = lax.bitcast_convert_type(
        jnp.int32(0x5F3759DF) - lax.shift_right_arithmetic(i, jnp.int32(1)),
        jnp.float32)
    for _ in range(3):
        y = y * (jnp.float32(1.5) - jnp.float32(0.5) * x * y * y)
    return y


def _bias_renorm(b):
    ab = jnp.abs(b)
    return b * jnp.where(ab > 2.0, 2.0 / (ab + 1e-7), 1.0)


_mesh = plsc.VectorSubcoreMesh(core_axis_name="c", subcore_axis_name="s")


@functools.partial(
    pl.kernel,
    mesh=_mesh,
    compiler_params=pltpu.CompilerParams(needs_layout_passes=False),
    out_type=jax.ShapeDtypeStruct((B,), jnp.float32),
    scratch_types=[
        pltpu.VMEM((ROWS_PER_W,), jnp.int32),
        pltpu.VMEM((ROWS_PER_W,), jnp.int32),
        pltpu.VMEM((NBUF, CHUNK, D), jnp.float32),
        pltpu.VMEM((NBUF, CHUNK, D), jnp.float32),
        pltpu.VMEM((NBUF, CHUNK), jnp.float32),
        pltpu.VMEM((NBUF, CHUNK), jnp.float32),
        pltpu.VMEM((2, CHUNK), jnp.float32),
        pltpu.SemaphoreType.DMA,
        pltpu.SemaphoreType.DMA,
        pltpu.SemaphoreType.DMA,
        pltpu.SemaphoreType.DMA,
        pltpu.SemaphoreType.DMA,
    ],
)
def _sc_predict(uf_hbm, mf_hbm, ub_hbm, mb_hbm, users_hbm, movies_hbm,
                out_hbm, uidx, midx, ubuf, mbuf, ubv, mbv, obuf,
                sem0, sem1, sem2, sem3, osem):
    wid = lax.axis_index("s") * NC + lax.axis_index("c")
    base = wid * ROWS_PER_W
    sems = (sem0, sem1, sem2, sem3)

    icp1 = pltpu.async_copy(users_hbm.at[pl.ds(base, ROWS_PER_W)], uidx, osem)
    icp2 = pltpu.async_copy(movies_hbm.at[pl.ds(base, ROWS_PER_W)], midx, osem)
    icp1.wait()
    icp2.wait()

    def fire(c):
        s = c % NBUF
        ui = uidx.at[pl.ds(c * CHUNK, CHUNK)]
        mi = midx.at[pl.ds(c * CHUNK, CHUNK)]
        return [
            pltpu.async_copy(uf_hbm.at[ui], ubuf.at[s], sems[s]),
            pltpu.async_copy(mf_hbm.at[mi], mbuf.at[s], sems[s]),
            pltpu.async_copy(ub_hbm.at[ui], ubv.at[s], sems[s]),
            pltpu.async_copy(mb_hbm.at[mi], mbv.at[s], sems[s]),
        ]

    inflight = [fire(0), fire(1), fire(2)]
    out_cps = []
    for c in range(NCHUNK):
        s = c % NBUF
        for cp in inflight.pop(0):
            cp.wait()
        if c + 3 < NCHUNK:
            inflight.append(fire(c + 3))
        if c >= 2:
            out_cps[c - 2].wait()

        ub2 = ubuf.at[s]
        mb2 = mbuf.at[s]
        lanei = lax.broadcasted_iota(jnp.int32, (L,), 0)
        rowss = [jnp.int32(g * L) + lanei for g in range(NG)]
        zero = jnp.zeros((L,), jnp.float32)

        def col_body(j, carry):
            cols = jnp.bitwise_and(lanei + j, jnp.int32(D - 1))
            out = []
            for g in range(NG):
                ad, au, am = carry[g]
                u = plsc.load_gather(ub2, [rowss[g], cols])
                m = plsc.load_gather(mb2, [rowss[g], cols])
                out.append((ad + u * m, au + u * u, am + m * m))
            return tuple(out)

        accs = lax.fori_loop(0, D, col_body,
                             tuple((zero, zero, zero) for _ in range(NG)))

        for g in range(NG):
            ad, au, am = accs[g]
            q = jnp.maximum(au * am, jnp.float32(1e-30))
            inv = _rsqrt(q)
            p = q * inv
            cos = jnp.where(p >= 1e-8, ad * inv, ad * jnp.float32(1e8))
            bu = _bias_renorm(ubv[s, pl.ds(g * L, L)])
            bm = _bias_renorm(mbv[s, pl.ds(g * L, L)])
            pred = (cos + bu + bm) * jnp.float32(2.25) + jnp.float32(2.75)
            obuf[c % 2, pl.ds(g * L, L)] = jnp.clip(pred, 0.5, 5.0)

        out_cps.append(pltpu.async_copy(
            obuf.at[c % 2], out_hbm.at[pl.ds(base + c * CHUNK, CHUNK)], osem))
    for cp in out_cps[-2:]:
        cp.wait()


def kernel(user_factors, movie_factors, user_biases, movie_biases, users, movies):
    return _sc_predict(user_factors, movie_factors,
                       user_biases[:, 0], movie_biases[:, 0],
                       users, movies)
